# baseline (speedup 1.0000x reference)
import functools

import jax
import jax.numpy as jnp
from jax import lax
from jax.experimental import pallas as pl
from jax.experimental.pallas import tpu as pltpu
from jax.experimental.pallas import tpu_sc as plsc

_B = 16384
_D = 32
_NC = 2
_NS = 16
_NW = _NC * _NS
_BPW = _B // _NW
_C = 16
_NCH = _BPW // _C

_mesh = plsc.VectorSubcoreMesh(core_axis_name="c", subcore_axis_name="s")


@functools.partial(
    pl.kernel,
    out_type=(
        jax.ShapeDtypeStruct((_B, _D), jnp.float32),
        jax.ShapeDtypeStruct((_B, _D), jnp.float32),
        jax.ShapeDtypeStruct((_B, _D), jnp.float32),
    ),
    mesh=_mesh,
    scratch_types=[
        pltpu.VMEM((_BPW,), jnp.int32),
        pltpu.VMEM((_BPW,), jnp.int32),
        pltpu.VMEM((2, _C, 8, _D), jnp.float32),
        pltpu.VMEM((_BPW, _D), jnp.float32),
        pltpu.SemaphoreType.DMA,
        pltpu.SemaphoreType.DMA,
        pltpu.SemaphoreType.DMA,
        pltpu.SemaphoreType.DMA,
        pltpu.SemaphoreType.DMA,
    ],
)
def _gather3(src_t, edge_t, dst_t, src_i, edge_i, dst_i,
             src_o, edge_o, dst_o,
             idx_v, gidx_v, rows_b, out_v, g0, g1, g2, g3, osem):
    wid = lax.axis_index("s") * _NC + lax.axis_index("c")
    base = wid * _BPW

    def one_table(table, ids, out):
        pltpu.sync_copy(ids.at[pl.ds(base, _BPW)], idx_v)
        tv = table.reshape(table.shape[0] // 8, 8, _D)

        @pl.loop(0, _BPW // 16)
        def gidx(c):
            gidx_v[pl.ds(c * 16, 16)] = lax.shift_right_logical(
                idx_v[pl.ds(c * 16, 16)], 3)

        gsems = [g0, g1]

        def gather(c, phase):
            vec_g = gidx_v[pl.ds(c * _C, _C)]
            for j in range(_C):
                pltpu.make_async_copy(
                    tv.at[vec_g[j]], rows_b.at[phase, j], gsems[phase]).start()

        def drain(phase):
            pltpu.make_async_copy(
                tv.at[pl.ds(0, _C)], rows_b.at[phase], gsems[phase]).wait()

        def extract(c, p):
            cb = c * _C
            vec = idx_v[pl.ds(cb, _C)]
            for j in range(_C):
                slot = lax.rem(vec[j], 8)
                out_v[cb + j, pl.ds(0, 16)] = rows_b[p, j, slot, pl.ds(0, 16)]
                out_v[cb + j, pl.ds(16, 16)] = rows_b[p, j, slot, pl.ds(16, 16)]

        gather(0, 0)

        @pl.loop(0, _NCH // 2)
        def pair(h):
            c0 = h * 2
            c1 = c0 + 1
            gather(c1, 1)
            drain(0)
            extract(c0, 0)

            @pl.when(c1 + 1 < _NCH)
            def _():
                gather(c1 + 1, 0)

            drain(1)
            extract(c1, 1)

        pltpu.async_copy(out_v, out.at[pl.ds(base, _BPW)], osem).wait()

    one_table(src_t, src_i, src_o)
    one_table(edge_t, edge_i, edge_o)
    one_table(dst_t, dst_i, dst_o)


def kernel(src_table, edge_table, dst_table, src_ids, edge_ids, dst_ids, offset):
    src_emb, edge_emb, dst_emb = _gather3(
        src_table, edge_table, dst_table,
        src_ids.astype(jnp.int32),
        edge_ids.astype(jnp.int32),
        dst_ids.astype(jnp.int32),
    )
    return (src_emb, edge_emb, dst_emb, offset)


# speedup vs baseline: 1.0556x; 1.0556x over previous
import functools

import jax
import jax.numpy as jnp
from jax import lax
from jax.experimental import pallas as pl
from jax.experimental.pallas import tpu as pltpu
from jax.experimental.pallas import tpu_sc as plsc

_B = 16384
_D = 32
_NC = 2
_NS = 16
_NW = _NC * _NS
_BPW = _B // _NW
_HALF = _BPW // 2
_C = 16
_SNCH = _HALF // _C

_mesh = plsc.VectorSubcoreMesh(core_axis_name="c", subcore_axis_name="s")


@functools.partial(
    pl.kernel,
    out_type=(
        jax.ShapeDtypeStruct((_B, _D), jnp.float32),
        jax.ShapeDtypeStruct((_B, _D), jnp.float32),
        jax.ShapeDtypeStruct((_B, _D), jnp.float32),
    ),
    mesh=_mesh,
    scratch_types=[
        pltpu.VMEM((_BPW,), jnp.int32),
        pltpu.VMEM((_HALF,), jnp.int32),
        pltpu.VMEM((2, _C, 8, _D), jnp.float32),
        pltpu.VMEM((_BPW, _D), jnp.float32),
        pltpu.SemaphoreType.DMA,
        pltpu.SemaphoreType.DMA,
        pltpu.SemaphoreType.DMA,
        pltpu.SemaphoreType.DMA,
    ],
)
def _gather3(src_t, edge_t, dst_t, src_i, edge_i, dst_i,
             src_o, edge_o, dst_o,
             idx_v, gidx_v, rows_b, out_v, g0, g1, dsem, osem):
    wid = lax.axis_index("s") * _NC + lax.axis_index("c")
    base = wid * _BPW

    def one_table(table, ids, out):
        pltpu.sync_copy(ids.at[pl.ds(base, _BPW)], idx_v)
        tv = table.reshape(table.shape[0] // 8, 8, _D)

        @pl.loop(0, _HALF // 16)
        def gidx(c):
            gidx_v[pl.ds(c * 16, 16)] = lax.shift_right_logical(
                idx_v[pl.ds(c * 16, 16)], 3)

        @pl.loop(0, _HALF // 16)
        def dmafire(c):
            cb = _HALF + c * 16
            vec = idx_v[pl.ds(cb, 16)]
            for j in range(16):
                pltpu.make_async_copy(
                    table.at[vec[j]], out_v.at[cb + j], dsem).start()

        gsems = [g0, g1]

        def gather(c, phase):
            vec_g = gidx_v[pl.ds(c * _C, _C)]
            for j in range(_C):
                pltpu.make_async_copy(
                    tv.at[vec_g[j]], rows_b.at[phase, j], gsems[phase]).start()

        def drain(phase):
            pltpu.make_async_copy(
                tv.at[pl.ds(0, _C)], rows_b.at[phase], gsems[phase]).wait()

        def extract(c, p):
            cb = c * _C
            vec = idx_v[pl.ds(cb, _C)]
            for j in range(_C):
                slot = lax.rem(vec[j], 8)
                out_v[cb + j, pl.ds(0, 16)] = rows_b[p, j, slot, pl.ds(0, 16)]
                out_v[cb + j, pl.ds(16, 16)] = rows_b[p, j, slot, pl.ds(16, 16)]

        gather(0, 0)

        @pl.loop(0, _SNCH // 2)
        def pair(h):
            c0 = h * 2
            c1 = c0 + 1
            gather(c1, 1)
            drain(0)
            extract(c0, 0)

            @pl.when(c1 + 1 < _SNCH)
            def _():
                gather(c1 + 1, 0)

            drain(1)
            extract(c1, 1)

        @pl.loop(0, _HALF // 16)
        def dmadrain(c):
            cb = _HALF + c * 16
            vec = idx_v[pl.ds(cb, 16)]
            for j in range(16):
                pltpu.make_async_copy(
                    table.at[vec[j]], out_v.at[cb + j], dsem).wait()

        pltpu.async_copy(out_v, out.at[pl.ds(base, _BPW)], osem).wait()

    one_table(src_t, src_i, src_o)
    one_table(edge_t, edge_i, edge_o)
    one_table(dst_t, dst_i, dst_o)


def kernel(src_table, edge_table, dst_table, src_ids, edge_ids, dst_ids, offset):
    src_emb, edge_emb, dst_emb = _gather3(
        src_table, edge_table, dst_table,
        src_ids.astype(jnp.int32),
        edge_ids.astype(jnp.int32),
        dst_ids.astype(jnp.int32),
    )
    return (src_emb, edge_emb, dst_emb, offset)
